# baseline (speedup 1.0000x reference)
import functools

import jax
import jax.numpy as jnp
from jax import lax
from jax.experimental import pallas as pl
from jax.experimental.pallas import tpu as pltpu
from jax.experimental.pallas import tpu_sc as plsc

_N = 10000
_E = 320000
_D = 128
_DOUT = 16

_NC = 2
_NS = 16
_CHUNK = 80
_NCH = 125
_EPT = _NCH * _CHUNK
_SLICE = 640

_BR = 1000
_NB = _N // _BR

_WIN = 136
_EROWS = _E // _CHUNK
_EROWS_PAD = _EROWS + 8


def _sc_mesh():
    return plsc.VectorSubcoreMesh(core_axis_name="c", subcore_axis_name="s")


def _zero_init(buf, accum, s):
    @pl.loop(0, _CHUNK)
    def _(r):
        @pl.loop(0, _D // 16)
        def _(q):
            buf[r, pl.ds(q * 16, 16)] = jnp.zeros((16,), jnp.float32)

    @pl.when(s < _NS - 1)
    def _():
        @pl.loop(0, _SLICE // _CHUNK)
        def _(i):
            pltpu.sync_copy(buf, accum.at[pl.ds(s * _SLICE + i * _CHUNK,
                                                _CHUNK)])

    @pl.when(s == _NS - 1)
    def _():
        @pl.loop(0, (_N - (_NS - 1) * _SLICE) // _CHUNK)
        def _(i):
            pltpu.sync_copy(buf, accum.at[pl.ds((_NS - 1) * _SLICE
                                                + i * _CHUNK, _CHUNK)])


def _drain(accum, out_hbm, c, s):
    tailrows = _N - (_NS - 1) * _SLICE

    @pl.when(s < _NS - 1)
    def _():
        pltpu.sync_copy(accum.at[pl.ds(s * _SLICE, _SLICE)],
                        out_hbm.at[pl.ds(c * _N + s * _SLICE, _SLICE)])

    @pl.when(s == _NS - 1)
    def _():
        pltpu.sync_copy(accum.at[pl.ds((_NS - 1) * _SLICE, tailrows)],
                        out_hbm.at[pl.ds(c * _N + (_NS - 1) * _SLICE,
                                         tailrows)])


def _sc_degree(dst3):
    @functools.partial(
        pl.kernel,
        mesh=_sc_mesh(),
        out_type=jax.ShapeDtypeStruct((_NC * _N, _D), jnp.float32),
        scratch_types=[
            pltpu.VMEM((_WIN, _CHUNK), jnp.int32),
            pltpu.VMEM((_CHUNK, _D), jnp.float32),
            pltpu.VMEM_SHARED((_N, _D), jnp.float32),
            pltpu.SemaphoreType.DMA,
        ],
    )
    def k(dst_hbm, out_hbm, didx, ones_v, accum, sem):
        c = lax.axis_index("c")
        s = lax.axis_index("s")
        t = c * _NS + s
        _zero_init(ones_v, accum, s)

        @pl.loop(0, _CHUNK)
        def _(r):
            @pl.loop(0, _D // 16)
            def _(q):
                ones_v[r, pl.ds(q * 16, 16)] = jnp.ones((16,), jnp.float32)

        start = t * _NCH
        start8 = (start // 8) * 8
        off = start - start8
        pltpu.sync_copy(dst_hbm.at[pl.ds(start8, _WIN)], didx)
        plsc.subcore_barrier()

        depth = 8

        @pl.loop(0, depth)
        def _(j):
            pltpu.async_copy(ones_v, accum.at[didx.at[off + j]], sem, add=True)

        @pl.loop(depth, _NCH)
        def _(j):
            pltpu.make_async_copy(ones_v, accum.at[didx.at[off + j]], sem).wait()
            pltpu.async_copy(ones_v, accum.at[didx.at[off + j]], sem, add=True)

        @pl.loop(0, depth)
        def _(j):
            pltpu.make_async_copy(ones_v, accum.at[didx.at[off + j]], sem).wait()

        plsc.subcore_barrier()
        _drain(accum, out_hbm, c, s)

    return k(dst3)


_NBUF = 2
_NGRP = (_NCH - 1) // _NBUF


def _sc_aggregate(y, ei, dst3):
    @functools.partial(
        pl.kernel,
        mesh=_sc_mesh(),
        out_type=jax.ShapeDtypeStruct((_NC * _N, _D), jnp.float32),
        scratch_types=[
            pltpu.VMEM((_EPT,), jnp.int32),
            pltpu.VMEM((_WIN, _CHUNK), jnp.int32),
            pltpu.VMEM((_NBUF, _CHUNK, _D), jnp.float32),
            pltpu.VMEM_SHARED((_N, _D), jnp.float32),
            pltpu.SemaphoreType.DMA((_NBUF,)),
            pltpu.SemaphoreType.DMA((_NBUF,)),
        ],
    )
    def k(y_hbm, ei_hbm, dst_hbm, out_hbm, sidx, didx, rows, accum,
          gsem, ssem):
        c = lax.axis_index("c")
        s = lax.axis_index("s")
        t = c * _NS + s
        _zero_init(rows.at[0], accum, s)
        start = t * _NCH
        start8 = (start // 8) * 8
        off = start - start8
        pltpu.sync_copy(ei_hbm.at[pl.ds(t * _EPT, _EPT)], sidx)
        pltpu.sync_copy(dst_hbm.at[pl.ds(start8, _WIN)], didx)
        plsc.subcore_barrier()

        def src_at(cjdx):
            return sidx.at[pl.ds(cjdx * _CHUNK, _CHUNK)]

        @pl.loop(0, _NGRP)
        def _(g):
            base = g * _NBUF
            for b in range(_NBUF):
                @pl.when(g > 0)
                def _():
                    pltpu.make_async_copy(
                        rows.at[b], accum.at[didx.at[off + base + b]],
                        ssem.at[b]).wait()

                pltpu.async_copy(y_hbm.at[src_at(base + b)],
                                 rows.at[b], gsem.at[b])
            for b in range(_NBUF):
                pltpu.make_async_copy(y_hbm.at[src_at(base + b)],
                                      rows.at[b], gsem.at[b]).wait()
                pltpu.async_copy(rows.at[b], accum.at[didx.at[off + base + b]],
                                 ssem.at[b], add=True)

        for b in range(_NBUF):
            pltpu.make_async_copy(rows.at[b], accum.at[didx.at[off + b]],
                                  ssem.at[b]).wait()

        last = _NGRP * _NBUF
        pltpu.sync_copy(y_hbm.at[src_at(last)], rows.at[0])
        pltpu.sync_copy(rows.at[0], accum.at[didx.at[off + last]], add=True)

        plsc.subcore_barrier()
        _drain(accum, out_hbm, c, s)

    return k(y, ei, dst3)


def _row_spec(width=_D):
    return pl.BlockSpec((_BR, width), lambda i: (i, 0))


def _part_specs(width=_D):
    return (pl.BlockSpec((_BR, width), lambda i: (i, 0)),
            pl.BlockSpec((_BR, width), lambda i: (i + _NB, 0)))


def _full(shape):
    return pl.BlockSpec(shape, lambda i: (0,) * len(shape))


def _tc_first(x, W1, degp):
    def body(x_ref, w_ref, p0_ref, p1_ref, y_ref, d_ref):
        dinv = lax.rsqrt(p0_ref[:, 0:1] + p1_ref[:, 0:1] + 1.0)
        d_ref[...] = jnp.broadcast_to(dinv, (_BR, 16))
        y_ref[...] = dinv * jnp.dot(x_ref[...], w_ref[...],
                                    preferred_element_type=jnp.float32)

    sp0, sp1 = _part_specs()
    return pl.pallas_call(
        body,
        grid=(_NB,),
        in_specs=[_row_spec(), _full((_D, _D)), sp0, sp1],
        out_specs=(_row_spec(), pl.BlockSpec((_BR, 16), lambda i: (i, 0))),
        out_shape=(jax.ShapeDtypeStruct((_N, _D), jnp.float32),
                   jax.ShapeDtypeStruct((_N, 16), jnp.float32)),
    )(x, W1, degp, degp)


def _tc_mid(aggp, y, dinv16, b, W, mask=None):
    def body(*refs):
        if mask is None:
            a0, a1, y_ref, d_ref, b_ref, w_ref, o_ref = refs
        else:
            a0, a1, y_ref, d_ref, b_ref, w_ref, m_ref, o_ref = refs
        dinv = d_ref[:, 0:1]
        h = jnp.tanh(dinv * (a0[...] + a1[...] + y_ref[...]) + b_ref[...])
        if mask is not None:
            h = h * m_ref[...]
        o_ref[...] = dinv * jnp.dot(h, w_ref[...],
                                    preferred_element_type=jnp.float32)

    a0s, a1s = _part_specs()
    dspec = pl.BlockSpec((_BR, 16), lambda i: (i, 0))
    in_specs = [a0s, a1s, _row_spec(), dspec, _full((1, _D)), _full((_D, _D))]
    args = [aggp, aggp, y, dinv16, b.reshape(1, _D), W]
    if mask is not None:
        in_specs.append(_row_spec())
        args.append(mask)
    return pl.pallas_call(
        body,
        grid=(_NB,),
        in_specs=in_specs,
        out_specs=_row_spec(),
        out_shape=jax.ShapeDtypeStruct((_N, _D), jnp.float32),
    )(*args)


def _tc_last(aggp, y, dinv16, b3, Wc, bc):
    def body(a0, a1, y_ref, d_ref, b_ref, wc_ref, bc_ref, h_ref, o_ref):
        dinv = d_ref[:, 0:1]
        h = jnp.tanh(dinv * (a0[...] + a1[...] + y_ref[...]) + b_ref[...])
        h_ref[...] = h
        o_ref[...] = jnp.dot(h, wc_ref[...],
                             preferred_element_type=jnp.float32) + bc_ref[...]

    a0s, a1s = _part_specs()
    dspec = pl.BlockSpec((_BR, 16), lambda i: (i, 0))
    return pl.pallas_call(
        body,
        grid=(_NB,),
        in_specs=[a0s, a1s, _row_spec(), dspec, _full((1, _D)),
                  _full((_D, _DOUT)), _full((1, _DOUT))],
        out_specs=(pl.BlockSpec((_BR, _D), lambda i: (i, 0)),
                   pl.BlockSpec((_BR, _DOUT), lambda i: (i, 0))),
        out_shape=(jax.ShapeDtypeStruct((_N, _D), jnp.float32),
                   jax.ShapeDtypeStruct((_N, _DOUT), jnp.float32)),
    )(aggp, aggp, y, dinv16, b3.reshape(1, _D), Wc, bc.reshape(1, _DOUT))


def kernel(x, edge_index, W1, b1, W2, b2, W3, b3, Wc, bc):
    f32 = jnp.float32
    src1 = edge_index[0]
    dst3 = jnp.pad(edge_index[1].reshape(_EROWS, _CHUNK),
                   ((0, _EROWS_PAD - _EROWS), (0, 0)))
    keep = jax.random.bernoulli(jax.random.key(42), 0.8, (_N, _D))
    mask = keep.astype(f32) / 0.8

    degp = _sc_degree(dst3)

    y1, dinv16 = _tc_first(x, W1, degp)
    agg1 = _sc_aggregate(y1, src1, dst3)
    y2 = _tc_mid(agg1, y1, dinv16, b1, W2, mask=mask)

    agg2 = _sc_aggregate(y2, src1, dst3)
    y3 = _tc_mid(agg2, y2, dinv16, b2, W3)

    agg3 = _sc_aggregate(y3, src1, dst3)
    h3, out = _tc_last(agg3, y3, dinv16, b3, Wc, bc)

    return out, h3
